# baseline (speedup 1.0000x reference)
import functools

import numpy as np
import jax
import jax.numpy as jnp
from jax import lax
from jax.experimental import pallas as pl
from jax.experimental.pallas import tpu as pltpu
from jax.experimental.pallas import tpu_sc as plsc

_N_LEVELS = 16
_T = 1 << 19
_MASK = np.int32(_T - 1)
_P1 = np.int32(-1640531535)
_P2 = np.int32(805459861)
_RES = [int(np.floor(16 * (1.5 ** l))) for l in range(_N_LEVELS)]
_NC, _NS = 2, 16
_NW = _NC * _NS
_N = 262144
_PPW = _N // _NW


def _encode_call(coords_flat, table_flat, tpari, tparf, three_d):
    NT = 2 if three_d else 1
    C = 64 if three_d else 128
    G = C // 16
    R = _N_LEVELS * NT
    EL = 8 * C
    SL = R * EL
    NCH = _PPW // C
    NP = NCH // 2
    D = 3 if three_d else 2
    dense = [(r + 1) ** D <= _T for r in _RES]
    mesh = plsc.VectorSubcoreMesh(core_axis_name="c", subcore_axis_name="s")

    scratch = [
        pltpu.VMEM((2 * _PPW,), jnp.float32),
        pltpu.VMEM((SL,), jnp.int32),
        pltpu.VMEM((SL,), jnp.int32),
        pltpu.VMEM((2, R, 4 * C), jnp.float32),
        pltpu.VMEM((SL,), jnp.float32),
        pltpu.VMEM((SL,), jnp.float32),
        pltpu.VMEM((32, 128), jnp.float32),
        pltpu.SemaphoreType.DMA,
        pltpu.SemaphoreType.DMA,
        pltpu.SemaphoreType.DMA,
    ]
    if three_d:
        scratch += [pltpu.VMEM((2 * _N_LEVELS, 16), jnp.int32),
                    pltpu.VMEM((2 * _N_LEVELS, 16), jnp.float32)]

    def body(*args):
        if three_d:
            (coords_hbm, tpari_hbm, tparf_hbm, table_hbm, out_hbm,
             cv, idxv0, idxv1, wv, rowsv0, rowsv1, encv,
             gsem0, gsem1, osem, tpiv, tpfv) = args
        else:
            (coords_hbm, table_hbm, out_hbm,
             cv, idxv0, idxv1, wv, rowsv0, rowsv1, encv,
             gsem0, gsem1, osem) = args
        gsems = (gsem0, gsem1)
        idxvs = (idxv0, idxv1)
        rowsvs = (rowsv0, rowsv1)
        wid = lax.axis_index("s") * _NC + lax.axis_index("c")
        base = wid * _PPW
        pltpu.sync_copy(coords_hbm.at[pl.ds(base * 2, _PPW * 2)], cv)
        if three_d:
            pltpu.sync_copy(tpari_hbm, tpiv)
            pltpu.sync_copy(tparf_hbm, tpfv)

        def gen_fire(ci, b):
            cb = ci * C

            def grp(g, carry):
                goff = g * 16
                coff = (cb // 128) * 256 + (cb % 128) + goff
                xg = cv[pl.ds(coff, 16)]
                yg = cv[pl.ds(coff + 128, 16)]
                for l in range(_N_LEVELS):
                    res = _RES[l]
                    s = res + 1
                    L20 = l << 20
                    px = xg * res
                    py = yg * res
                    ix = jnp.clip(px.astype(jnp.int32), 0, res - 1)
                    iy = jnp.clip(py.astype(jnp.int32), 0, res - 1)
                    fx = px - ix.astype(jnp.float32)
                    fy = py - iy.astype(jnp.float32)
                    wx0 = 1.0 - fx
                    wy0 = 1.0 - fy
                    w4 = (wx0 * wy0, fx * wy0, wx0 * fy, fx * fy)
                    if dense[l]:
                        b00 = ix + iy * s
                        cidx = (b00, b00 + 1, b00 + s, b00 + s + 1)
                    else:
                        hy0 = iy * _P1
                        hy1 = hy0 + _P1
                        cidx = (ix ^ hy0, (ix + 1) ^ hy0,
                                ix ^ hy1, (ix + 1) ^ hy1)
                    for tc in range(NT):
                        r = l * NT + tc
                        if three_d:
                            ct = tpiv[2 * l + tc, :]
                            wt = tpfv[2 * l + tc, :]
                        for c in range(4):
                            if three_d:
                                if dense[l]:
                                    cell = cidx[c] + ct
                                else:
                                    cell = (cidx[c] ^ ct) & _MASK
                                wc = w4[c] * wt
                            else:
                                if dense[l]:
                                    cell = cidx[c]
                                else:
                                    cell = cidx[c] & _MASK
                                wc = w4[c]
                            ph = (cell + lax.shift_left(
                                lax.shift_right_logical(cell, 7), 7)) + L20
                            po = r * EL + c * C + goff
                            idxvs[b][pl.ds(po, 16)] = ph
                            idxvs[b][pl.ds(po + 4 * C, 16)] = ph + 128
                            wv[b, r, pl.ds(c * C + goff, 16)] = wc
                return carry

            lax.fori_loop(0, G, grp, 0)
            pltpu.async_copy(table_hbm.at[idxvs[b]], rowsvs[b], gsems[b])

        def wait_gathers(b):
            pltpu.make_async_copy(
                table_hbm.at[idxvs[b]], rowsvs[b], gsems[b]).wait()

        bbase = wid * (_PPW // 128)

        def drain_out():
            pltpu.make_async_copy(
                encv, out_hbm.at[bbase], osem).wait()

        def accum(ci, b, sub0):
            def grp(g, carry):
                goff = g * 16
                so = sub0 + goff
                for l in range(_N_LEVELS):
                    wrows = [[wv[b, l * NT + tc, pl.ds(c * C + goff, 16)]
                              for c in range(4)] for tc in range(NT)]
                    for f in range(2):
                        acc = None
                        for tc in range(NT):
                            rb = (l * NT + tc) * EL + f * 4 * C
                            for c in range(4):
                                vals = rowsvs[b][pl.ds(rb + c * C + goff, 16)]
                                term = vals * wrows[tc][c]
                                acc = term if acc is None else acc + term
                        encv[2 * l + f, pl.ds(so, 16)] = acc
                return carry

            lax.fori_loop(0, G, grp, 0)

        gen_fire(0, 0)

        if C == 128:
            def pair(i2, carry):
                i = i2 * 2
                gen_fire(i + 1, 1)
                wait_gathers(0)
                pl.when(i2 > 0)(drain_out)
                accum(i, 0, 0)
                pltpu.async_copy(encv, out_hbm.at[bbase + i], osem)
                pl.when(i2 < NP - 1)(lambda: gen_fire(i + 2, 0))
                wait_gathers(1)
                drain_out()
                accum(i + 1, 1, 0)
                pltpu.async_copy(encv, out_hbm.at[bbase + i + 1], osem)
                return carry
        else:
            def pair(i2, carry):
                i = i2 * 2
                gen_fire(i + 1, 1)
                wait_gathers(0)
                pl.when(i2 > 0)(drain_out)
                accum(i, 0, 0)
                pl.when(i2 < NP - 1)(lambda: gen_fire(i + 2, 0))
                wait_gathers(1)
                accum(i + 1, 1, 64)
                pltpu.async_copy(encv, out_hbm.at[bbase + i2], osem)
                return carry

        lax.fori_loop(0, NP, pair, 0)
        drain_out()

    kern = pl.kernel(body,
                     out_type=jax.ShapeDtypeStruct((_N // 128, 32, 128),
                                                   jnp.float32),
                     mesh=mesh, scratch_types=scratch,
                     compiler_params=pltpu.CompilerParams(
                         needs_layout_passes=False))
    if three_d:
        return kern(coords_flat, tpari, tparf, table_flat)
    return kern(coords_flat, table_flat)


def _mlp_call(pe_blocks_in, f2, exrow, enc, w0c, w0e, w1, w2, n_out,
              is_deform):
    BLK = 2048
    grid_steps = _N // BLK
    BR = BLK // 128

    def body(fin_ref, ex_ref, enc_ref, w0c_ref, w0e_ref, w1_ref, w2_ref,
             out_ref):
        fin = fin_ref[...]
        f2b = fin.reshape(BR, 2, 128).swapaxes(1, 2).reshape(BLK, 2)
        encb = enc_ref[...].swapaxes(1, 2).reshape(BLK, 32)
        h = jnp.dot(encb, w0e_ref[...],
                    preferred_element_type=jnp.float32)
        h = h + jnp.dot(f2b, w0c_ref[...],
                        preferred_element_type=jnp.float32)
        h = h + ex_ref[...]
        h = jnp.maximum(h, 0.0)
        h = jnp.maximum(jnp.dot(h, w1_ref[...],
                                preferred_element_type=jnp.float32), 0.0)
        o = jnp.dot(h, w2_ref[...], preferred_element_type=jnp.float32)
        if is_deform:
            pe = (o / 5.0 + f2b + 0.3) / 1.6
            out_ref[...] = pe.reshape(BR, 128, 2).swapaxes(1, 2).reshape(
                2 * BR, 128)
        else:
            out_ref[...] = o

    if is_deform:
        out_shape = jax.ShapeDtypeStruct((_N // 128 * 2, 128), jnp.float32)
        out_spec = pl.BlockSpec((2 * BR, 128), lambda i: (i, 0))
    else:
        out_shape = jax.ShapeDtypeStruct((_N, n_out), jnp.float32)
        out_spec = pl.BlockSpec((BLK, n_out), lambda i: (i, 0))

    fin = pe_blocks_in if pe_blocks_in is not None else f2
    return pl.pallas_call(
        body,
        grid=(grid_steps,),
        in_specs=[
            pl.BlockSpec((2 * BR, 128), lambda i: (i, 0)),
            pl.BlockSpec((1, 64), lambda i: (0, 0)),
            pl.BlockSpec((BR, 32, 128), lambda i: (i, 0, 0)),
            pl.BlockSpec((2, 64), lambda i: (0, 0)),
            pl.BlockSpec((32, 64), lambda i: (0, 0)),
            pl.BlockSpec((64, 64), lambda i: (0, 0)),
            pl.BlockSpec((64, n_out), lambda i: (0, 0)),
        ],
        out_specs=out_spec,
        out_shape=out_shape,
    )(fin, exrow, enc, w0c, w0e, w1, w2)


def kernel(tseq, grid, deform_table, dW0, dW1, dW2, video_table, vW0, vW1,
           vW2):
    gblocks = grid.reshape(_N // 128, 128, 2).swapaxes(1, 2).reshape(
        _N // 128 * 2, 128)
    gflat = gblocks.reshape(-1)
    dtab = deform_table.reshape(_N_LEVELS, _T // 128, 128, 2).swapaxes(
        2, 3).reshape(-1)
    vtab = video_table.reshape(_N_LEVELS, _T // 128, 128, 2).swapaxes(
        2, 3).reshape(-1)

    t = tseq[0, 0]
    cti, ctf = [], []
    for l in range(_N_LEVELS):
        res = _RES[l]
        s = res + 1
        pt = t * res
        it0f = jnp.clip(jnp.floor(pt), 0.0, float(res - 1))
        ft = pt - it0f
        it0 = it0f.astype(jnp.int32)
        it1 = it0 + 1
        if s ** 3 <= _T:
            ct0 = it0 * (s * s)
            ct1 = it1 * (s * s)
        else:
            ct0 = it0 * _P2
            ct1 = it1 * _P2
        cti += [ct0, ct1]
        ctf += [1.0 - ft, ft]
    tpari = jnp.broadcast_to(jnp.stack(cti)[:, None],
                             (2 * _N_LEVELS, 16)).astype(jnp.int32)
    tparf = jnp.broadcast_to(jnp.stack(ctf)[:, None],
                             (2 * _N_LEVELS, 16)).astype(jnp.float32)

    enc1 = _encode_call(gflat, dtab, tpari, tparf, three_d=True)
    exrow = t * dW0[2:3, :]
    pe_blocks = _mlp_call(None, gblocks, exrow, enc1, dW0[:2], dW0[3:],
                          dW1, dW2, 2, is_deform=True)
    enc2 = _encode_call(pe_blocks.reshape(-1), vtab, None, None,
                        three_d=False)
    out = _mlp_call(pe_blocks, None, jnp.zeros((1, 64), jnp.float32), enc2,
                    vW0[:2], vW0[2:], vW1, vW2, 3, is_deform=False)
    return out


# speedup vs baseline: 2.3330x; 2.3330x over previous
import functools

import numpy as np
import jax
import jax.numpy as jnp
from jax import lax
from jax.experimental import pallas as pl
from jax.experimental.pallas import tpu as pltpu
from jax.experimental.pallas import tpu_sc as plsc

_N_LEVELS = 16
_T = 1 << 19
_MASK = np.int32(_T - 1)
_P1 = np.int32(-1640531535)
_P2 = np.int32(805459861)
_NM128 = np.int32(-128)
_RES = [int(np.floor(16 * (1.5 ** l))) for l in range(_N_LEVELS)]
_NC, _NS = 2, 16
_NW = _NC * _NS
_N = 262144
_PPW = _N // _NW


def _encode_call(coords_flat, table_flat, tpari, tparf, three_d):
    NT = 2 if three_d else 1
    C = 64
    G = C // 16
    SD = list(range(4)) if three_d else list(range(6))
    SLV = [l for l in range(_N_LEVELS) if l not in SD]
    ROWS = [(l, tc) for l in SLV for tc in range(NT)]
    RMAP = {lt: r for r, lt in enumerate(ROWS)}
    R = len(ROWS)
    EL = 8 * C
    SL = R * EL
    NCH = _PPW // C
    NP = NCH // 2
    D = 3 if three_d else 2
    dense = [(r + 1) ** D <= _T for r in _RES]
    if three_d:
        nblk = [((2 * (_RES[l] + 1) ** 2) >> 7) + 2 for l in SD]
    else:
        nblk = [(((_RES[l] + 1) ** 2) + 127) >> 7 for l in SD]
    dloff = [0] * len(SD)
    for i in range(1, len(SD)):
        dloff[i] = dloff[i - 1] + nblk[i - 1] * 256
    DTOT = dloff[-1] + nblk[-1] * 256
    mesh = plsc.VectorSubcoreMesh(core_axis_name="c", subcore_axis_name="s")

    scratch = [
        pltpu.VMEM((2 * _PPW,), jnp.float32),
        pltpu.VMEM((SL,), jnp.int32),
        pltpu.VMEM((SL,), jnp.int32),
        pltpu.VMEM((2, R, 4 * C), jnp.float32),
        pltpu.VMEM((SL,), jnp.float32),
        pltpu.VMEM((SL,), jnp.float32),
        pltpu.VMEM((32, 128), jnp.float32),
        pltpu.VMEM((DTOT,), jnp.float32),
        pltpu.SemaphoreType.DMA,
        pltpu.SemaphoreType.DMA,
        pltpu.SemaphoreType.DMA,
    ]
    if three_d:
        scratch += [pltpu.VMEM((2 * _N_LEVELS, 16), jnp.int32),
                    pltpu.VMEM((2 * _N_LEVELS, 16), jnp.float32)]

    def body(*args):
        if three_d:
            (coords_hbm, tpari_hbm, tparf_hbm, table_hbm, out_hbm,
             cv, idxv0, idxv1, wv, rowsv0, rowsv1, encv, densev,
             gsem0, gsem1, osem, tpiv, tpfv) = args
        else:
            (coords_hbm, table_hbm, out_hbm,
             cv, idxv0, idxv1, wv, rowsv0, rowsv1, encv, densev,
             gsem0, gsem1, osem) = args
        gsems = (gsem0, gsem1)
        idxvs = (idxv0, idxv1)
        rowsvs = (rowsv0, rowsv1)
        wid = lax.axis_index("s") * _NC + lax.axis_index("c")
        base = wid * _PPW
        pltpu.sync_copy(coords_hbm.at[pl.ds(base * 2, _PPW * 2)], cv)
        if three_d:
            pltpu.sync_copy(tpari_hbm, tpiv)
            pltpu.sync_copy(tparf_hbm, tpfv)
        sblk = []
        for i, l in enumerate(SD):
            if three_d:
                ct0 = tpiv[2 * l, :][0]
                sb = lax.shift_right_logical(ct0, 7)
                pltpu.sync_copy(
                    table_hbm.at[pl.ds((l << 20) + sb * 256, nblk[i] * 256)],
                    densev.at[pl.ds(dloff[i], nblk[i] * 256)])
                sblk.append(lax.shift_left(sb, 7))
            else:
                pltpu.sync_copy(
                    table_hbm.at[pl.ds(l << 20, nblk[i] * 256)],
                    densev.at[pl.ds(dloff[i], nblk[i] * 256)])
                sblk.append(0)

        def corners(xg, yg, l):
            res = _RES[l]
            s = res + 1
            px = xg * res
            py = yg * res
            ix = jnp.clip(px.astype(jnp.int32), 0, res - 1)
            iy = jnp.clip(py.astype(jnp.int32), 0, res - 1)
            fx = px - ix.astype(jnp.float32)
            fy = py - iy.astype(jnp.float32)
            wx0 = 1.0 - fx
            wy0 = 1.0 - fy
            w4 = (wx0 * wy0, fx * wy0, wx0 * fy, fx * fy)
            if dense[l]:
                b00 = ix + iy * s
                cidx = (b00, b00 + 1, b00 + s, b00 + s + 1)
            else:
                hy0 = iy * _P1
                hy1 = hy0 + _P1
                cidx = (ix ^ hy0, (ix + 1) ^ hy0, ix ^ hy1, (ix + 1) ^ hy1)
            return cidx, w4

        def gen_fire(ci, b):
            cb = ci * C

            def grp(g, carry):
                goff = g * 16
                coff = (cb // 128) * 256 + (cb % 128) + goff
                xg = cv[pl.ds(coff, 16)]
                yg = cv[pl.ds(coff + 128, 16)]
                for l in SLV:
                    L20 = l << 20
                    cidx, w4 = corners(xg, yg, l)
                    for tc in range(NT):
                        r = RMAP[(l, tc)]
                        if three_d:
                            ct = tpiv[2 * l + tc, :]
                            wt = tpfv[2 * l + tc, :]
                        for c in range(4):
                            if three_d:
                                if dense[l]:
                                    cell = cidx[c] + ct
                                else:
                                    cell = (cidx[c] ^ ct) & _MASK
                                wc = w4[c] * wt
                            else:
                                if dense[l]:
                                    cell = cidx[c]
                                else:
                                    cell = cidx[c] & _MASK
                                wc = w4[c]
                            ph = (cell + (cell & _NM128)) + L20
                            po = r * EL + c * C + goff
                            idxvs[b][pl.ds(po, 16)] = ph
                            idxvs[b][pl.ds(po + 4 * C, 16)] = ph + 128
                            wv[b, r, pl.ds(c * C + goff, 16)] = wc
                return carry

            lax.fori_loop(0, G, grp, 0)
            pltpu.async_copy(table_hbm.at[idxvs[b]], rowsvs[b], gsems[b])

        def wait_gathers(b):
            pltpu.make_async_copy(
                table_hbm.at[idxvs[b]], rowsvs[b], gsems[b]).wait()

        bbase = wid * (_PPW // 128)

        def drain_out():
            pltpu.make_async_copy(
                encv, out_hbm.at[bbase], osem).wait()

        def accum(ci, b, sub0):
            cb = ci * C

            def grp(g, carry):
                goff = g * 16
                so = sub0 + goff
                coff = (cb // 128) * 256 + (cb % 128) + goff
                xg = cv[pl.ds(coff, 16)]
                yg = cv[pl.ds(coff + 128, 16)]
                for i, l in enumerate(SD):
                    cidx, w4 = corners(xg, yg, l)
                    s = _RES[l] + 1
                    cdel = (1, s, s + 1)
                    accs = [None, None]
                    for tc in range(NT):
                        if three_d:
                            cell0 = cidx[0] + tpiv[2 * l + tc, :] - sblk[i]
                            wt = tpfv[2 * l + tc, :]
                        else:
                            cell0 = cidx[0]
                        for c in range(4):
                            cell = cell0 if c == 0 else cell0 + cdel[c - 1]
                            wc = w4[c] * wt if three_d else w4[c]
                            pos = dloff[i] + cell + (cell & _NM128)
                            for f in range(2):
                                vals = plsc.load_gather(
                                    densev, [pos + f * 128])
                                term = vals * wc
                                accs[f] = (term if accs[f] is None
                                           else accs[f] + term)
                    for f in range(2):
                        encv[2 * l + f, pl.ds(so, 16)] = accs[f]
                for l in SLV:
                    for f in range(2):
                        acc = None
                        for tc in range(NT):
                            r = RMAP[(l, tc)]
                            rb = r * EL + f * 4 * C
                            for c in range(4):
                                w = wv[b, r, pl.ds(c * C + goff, 16)]
                                vals = rowsvs[b][pl.ds(rb + c * C + goff, 16)]
                                term = vals * w
                                acc = term if acc is None else acc + term
                        encv[2 * l + f, pl.ds(so, 16)] = acc
                return carry

            lax.fori_loop(0, G, grp, 0)

        gen_fire(0, 0)

        def pair(i2, carry):
            i = i2 * 2
            gen_fire(i + 1, 1)
            wait_gathers(0)
            pl.when(i2 > 0)(drain_out)
            accum(i, 0, 0)
            pl.when(i2 < NP - 1)(lambda: gen_fire(i + 2, 0))
            wait_gathers(1)
            accum(i + 1, 1, 64)
            pltpu.async_copy(encv, out_hbm.at[bbase + i2], osem)
            return carry

        lax.fori_loop(0, NP, pair, 0)
        drain_out()

    kern = pl.kernel(body,
                     out_type=jax.ShapeDtypeStruct((_N // 128, 32, 128),
                                                   jnp.float32),
                     mesh=mesh, scratch_types=scratch,
                     compiler_params=pltpu.CompilerParams(
                         needs_layout_passes=False))
    if three_d:
        return kern(coords_flat, tpari, tparf, table_flat)
    return kern(coords_flat, table_flat)


def _mlp_call(pe_blocks_in, f2, exrow, enc, w0c, w0e, w1, w2, n_out,
              is_deform):
    BLK = 2048
    grid_steps = _N // BLK
    BR = BLK // 128

    def body(fin_ref, ex_ref, enc_ref, w0c_ref, w0e_ref, w1_ref, w2_ref,
             out_ref):
        fin = fin_ref[...]
        f2b = fin.reshape(BR, 2, 128).swapaxes(1, 2).reshape(BLK, 2)
        encb = enc_ref[...].swapaxes(1, 2).reshape(BLK, 32)
        h = jnp.dot(encb, w0e_ref[...],
                    preferred_element_type=jnp.float32)
        h = h + jnp.dot(f2b, w0c_ref[...],
                        preferred_element_type=jnp.float32)
        h = h + ex_ref[...]
        h = jnp.maximum(h, 0.0)
        h = jnp.maximum(jnp.dot(h, w1_ref[...],
                                preferred_element_type=jnp.float32), 0.0)
        o = jnp.dot(h, w2_ref[...], preferred_element_type=jnp.float32)
        if is_deform:
            pe = (o / 5.0 + f2b + 0.3) / 1.6
            out_ref[...] = pe.reshape(BR, 128, 2).swapaxes(1, 2).reshape(
                2 * BR, 128)
        else:
            out_ref[...] = o

    if is_deform:
        out_shape = jax.ShapeDtypeStruct((_N // 128 * 2, 128), jnp.float32)
        out_spec = pl.BlockSpec((2 * BR, 128), lambda i: (i, 0))
    else:
        out_shape = jax.ShapeDtypeStruct((_N, n_out), jnp.float32)
        out_spec = pl.BlockSpec((BLK, n_out), lambda i: (i, 0))

    fin = pe_blocks_in if pe_blocks_in is not None else f2
    return pl.pallas_call(
        body,
        grid=(grid_steps,),
        in_specs=[
            pl.BlockSpec((2 * BR, 128), lambda i: (i, 0)),
            pl.BlockSpec((1, 64), lambda i: (0, 0)),
            pl.BlockSpec((BR, 32, 128), lambda i: (i, 0, 0)),
            pl.BlockSpec((2, 64), lambda i: (0, 0)),
            pl.BlockSpec((32, 64), lambda i: (0, 0)),
            pl.BlockSpec((64, 64), lambda i: (0, 0)),
            pl.BlockSpec((64, n_out), lambda i: (0, 0)),
        ],
        out_specs=out_spec,
        out_shape=out_shape,
    )(fin, exrow, enc, w0c, w0e, w1, w2)


def kernel(tseq, grid, deform_table, dW0, dW1, dW2, video_table, vW0, vW1,
           vW2):
    gblocks = grid.reshape(_N // 128, 128, 2).swapaxes(1, 2).reshape(
        _N // 128 * 2, 128)
    gflat = gblocks.reshape(-1)
    dtab = deform_table.reshape(_N_LEVELS, _T // 128, 128, 2).swapaxes(
        2, 3).reshape(-1)
    vtab = video_table.reshape(_N_LEVELS, _T // 128, 128, 2).swapaxes(
        2, 3).reshape(-1)

    t = tseq[0, 0]
    cti, ctf = [], []
    for l in range(_N_LEVELS):
        res = _RES[l]
        s = res + 1
        pt = t * res
        it0f = jnp.clip(jnp.floor(pt), 0.0, float(res - 1))
        ft = pt - it0f
        it0 = it0f.astype(jnp.int32)
        it1 = it0 + 1
        if s ** 3 <= _T:
            ct0 = it0 * (s * s)
            ct1 = it1 * (s * s)
        else:
            ct0 = it0 * _P2
            ct1 = it1 * _P2
        cti += [ct0, ct1]
        ctf += [1.0 - ft, ft]
    tpari = jnp.broadcast_to(jnp.stack(cti)[:, None],
                             (2 * _N_LEVELS, 16)).astype(jnp.int32)
    tparf = jnp.broadcast_to(jnp.stack(ctf)[:, None],
                             (2 * _N_LEVELS, 16)).astype(jnp.float32)

    enc1 = _encode_call(gflat, dtab, tpari, tparf, three_d=True)
    exrow = t * dW0[2:3, :]
    pe_blocks = _mlp_call(None, gblocks, exrow, enc1, dW0[:2], dW0[3:],
                          dW1, dW2, 2, is_deform=True)
    enc2 = _encode_call(pe_blocks.reshape(-1), vtab, None, None,
                        three_d=False)
    out = _mlp_call(pe_blocks, None, jnp.zeros((1, 64), jnp.float32), enc2,
                    vW0[:2], vW0[2:], vW1, vW2, 3, is_deform=False)
    return out


# speedup vs baseline: 2.3922x; 1.0254x over previous
import functools

import numpy as np
import jax
import jax.numpy as jnp
from jax import lax
from jax.experimental import pallas as pl
from jax.experimental.pallas import tpu as pltpu
from jax.experimental.pallas import tpu_sc as plsc

_N_LEVELS = 16
_T = 1 << 19
_MASK = np.int32(_T - 1)
_P1 = np.int32(-1640531535)
_P2 = np.int32(805459861)
_NM128 = np.int32(-128)
_RES = [int(np.floor(16 * (1.5 ** l))) for l in range(_N_LEVELS)]
_NC, _NS = 2, 16
_NW = _NC * _NS
_N = 262144
_PPW = _N // _NW


def _encode_call(coords_flat, table_flat, tpari, tparf, three_d):
    NT = 2 if three_d else 1
    C = 64
    G = C // 16
    SD = list(range(4)) if three_d else list(range(6))
    BLD = [4] if three_d else []
    SLV = [l for l in range(_N_LEVELS)
           if l not in SD and l not in BLD]
    ROWS = [(l, tc) for l in SLV for tc in range(NT)]
    RMAP = {lt: r for r, lt in enumerate(ROWS)}
    R = len(ROWS)
    EL = 8 * C
    SL = R * EL
    NCH = _PPW // C
    NP = NCH // 2
    D = 3 if three_d else 2
    dense = [(r + 1) ** D <= _T for r in _RES]
    if three_d:
        nblk = [((2 * (_RES[l] + 1) ** 2) >> 7) + 2 for l in SD]
    else:
        nblk = [(((_RES[l] + 1) ** 2) + 127) >> 7 for l in SD]
    dloff = [0] * len(SD)
    for i in range(1, len(SD)):
        dloff[i] = dloff[i - 1] + nblk[i - 1] * 256
    DTOT = dloff[-1] + nblk[-1] * 256
    BOFF = DTOT
    if BLD:
        BS = _RES[BLD[0]] + 1
        BNCELL = BS * BS
        DTOT += (((BNCELL + 127) >> 7) + 1) * 256
    mesh = plsc.VectorSubcoreMesh(core_axis_name="c", subcore_axis_name="s")

    scratch = [
        pltpu.VMEM((2 * _PPW,), jnp.float32),
        pltpu.VMEM((SL,), jnp.int32),
        pltpu.VMEM((SL,), jnp.int32),
        pltpu.VMEM((2, R, 4 * C), jnp.float32),
        pltpu.VMEM((SL,), jnp.float32),
        pltpu.VMEM((SL,), jnp.float32),
        pltpu.VMEM((32, 128), jnp.float32),
        pltpu.VMEM((DTOT,), jnp.float32),
        pltpu.SemaphoreType.DMA,
        pltpu.SemaphoreType.DMA,
        pltpu.SemaphoreType.DMA,
    ]
    if three_d:
        scratch += [pltpu.VMEM((2 * _N_LEVELS, 16), jnp.int32),
                    pltpu.VMEM((2 * _N_LEVELS, 16), jnp.float32)]

    def body(*args):
        if three_d:
            (coords_hbm, tpari_hbm, tparf_hbm, table_hbm, out_hbm,
             cv, idxv0, idxv1, wv, rowsv0, rowsv1, encv, densev,
             gsem0, gsem1, osem, tpiv, tpfv) = args
        else:
            (coords_hbm, table_hbm, out_hbm,
             cv, idxv0, idxv1, wv, rowsv0, rowsv1, encv, densev,
             gsem0, gsem1, osem) = args
        gsems = (gsem0, gsem1)
        idxvs = (idxv0, idxv1)
        rowsvs = (rowsv0, rowsv1)
        wid = lax.axis_index("s") * _NC + lax.axis_index("c")
        base = wid * _PPW
        pltpu.sync_copy(coords_hbm.at[pl.ds(base * 2, _PPW * 2)], cv)
        if three_d:
            pltpu.sync_copy(tpari_hbm, tpiv)
            pltpu.sync_copy(tparf_hbm, tpfv)
        sblk = []
        for i, l in enumerate(SD):
            if three_d:
                ct0 = tpiv[2 * l, :][0]
                sb = lax.shift_right_logical(ct0, 7)
                pltpu.sync_copy(
                    table_hbm.at[pl.ds((l << 20) + sb * 256, nblk[i] * 256)],
                    densev.at[pl.ds(dloff[i], nblk[i] * 256)])
                sblk.append(lax.shift_left(sb, 7))
            else:
                pltpu.sync_copy(
                    table_hbm.at[pl.ds(l << 20, nblk[i] * 256)],
                    densev.at[pl.ds(dloff[i], nblk[i] * 256)])
                sblk.append(0)
        if BLD:
            BL = BLD[0]
            L20B = BL << 20
            CPC = SL // 4
            lanes = lax.iota(jnp.int32, 16)
            wt0 = tpfv[2 * BL, :]
            wt1 = tpfv[2 * BL + 1, :]
            cc0 = 0
            while cc0 < BNCELL:
                ncell = min(CPC, ((BNCELL - cc0 + 15) // 16) * 16)
                ngrp = ncell // 16

                def bgrp(g, carry):
                    gpos = g * 16
                    cvec = cc0 + gpos + lanes
                    yv = cvec // BS
                    xv = cvec - yv * BS
                    u = xv ^ (yv * _P1)
                    for tc in range(2):
                        cell = (u ^ tpiv[2 * BL + tc, :]) & _MASK
                        ph = (cell + (cell & _NM128)) + L20B
                        idxv0[pl.ds((tc * 2) * CPC + gpos, 16)] = ph
                        idxv0[pl.ds((tc * 2 + 1) * CPC + gpos, 16)] = ph + 128
                    return carry

                lax.fori_loop(0, ngrp, bgrp, 0)
                pltpu.async_copy(table_hbm.at[idxv0], rowsv0, gsem0).wait()

                def sgrp(g, carry):
                    gpos = g * 16
                    cc = cc0 + gpos
                    pos0 = BOFF + cc + (cc & -128)
                    for f in range(2):
                        v = (wt0 * rowsv0[pl.ds(f * CPC + gpos, 16)]
                             + wt1 * rowsv0[pl.ds((2 + f) * CPC + gpos, 16)])
                        densev[pl.ds(pos0 + f * 128, 16)] = v
                    return carry

                lax.fori_loop(0, ngrp, sgrp, 0)
                cc0 += CPC

        def corners(xg, yg, l, force_dense=False):
            res = _RES[l]
            s = res + 1
            px = xg * res
            py = yg * res
            ix = jnp.clip(px.astype(jnp.int32), 0, res - 1)
            iy = jnp.clip(py.astype(jnp.int32), 0, res - 1)
            fx = px - ix.astype(jnp.float32)
            fy = py - iy.astype(jnp.float32)
            wx0 = 1.0 - fx
            wy0 = 1.0 - fy
            w4 = (wx0 * wy0, fx * wy0, wx0 * fy, fx * fy)
            if dense[l] or force_dense:
                b00 = ix + iy * s
                cidx = (b00, b00 + 1, b00 + s, b00 + s + 1)
            else:
                hy0 = iy * _P1
                hy1 = hy0 + _P1
                cidx = (ix ^ hy0, (ix + 1) ^ hy0, ix ^ hy1, (ix + 1) ^ hy1)
            return cidx, w4

        def gen_fire(ci, b):
            cb = ci * C

            def grp(g, carry):
                goff = g * 16
                coff = (cb // 128) * 256 + (cb % 128) + goff
                xg = cv[pl.ds(coff, 16)]
                yg = cv[pl.ds(coff + 128, 16)]
                for l in SLV:
                    L20 = l << 20
                    cidx, w4 = corners(xg, yg, l)
                    for tc in range(NT):
                        r = RMAP[(l, tc)]
                        if three_d:
                            ct = tpiv[2 * l + tc, :]
                            wt = tpfv[2 * l + tc, :]
                        for c in range(4):
                            if three_d:
                                if dense[l]:
                                    cell = cidx[c] + ct
                                else:
                                    cell = (cidx[c] ^ ct) & _MASK
                                wc = w4[c] * wt
                            else:
                                if dense[l]:
                                    cell = cidx[c]
                                else:
                                    cell = cidx[c] & _MASK
                                wc = w4[c]
                            ph = (cell + (cell & _NM128)) + L20
                            po = r * EL + c * C + goff
                            idxvs[b][pl.ds(po, 16)] = ph
                            idxvs[b][pl.ds(po + 4 * C, 16)] = ph + 128
                            wv[b, r, pl.ds(c * C + goff, 16)] = wc
                return carry

            lax.fori_loop(0, G, grp, 0)
            pltpu.async_copy(table_hbm.at[idxvs[b]], rowsvs[b], gsems[b])

        def wait_gathers(b):
            pltpu.make_async_copy(
                table_hbm.at[idxvs[b]], rowsvs[b], gsems[b]).wait()

        bbase = wid * (_PPW // 128)

        def drain_out():
            pltpu.make_async_copy(
                encv, out_hbm.at[bbase], osem).wait()

        def accum(ci, b, sub0):
            cb = ci * C

            def grp(g, carry):
                goff = g * 16
                so = sub0 + goff
                coff = (cb // 128) * 256 + (cb % 128) + goff
                xg = cv[pl.ds(coff, 16)]
                yg = cv[pl.ds(coff + 128, 16)]
                for l in BLD:
                    cidx, w4 = corners(xg, yg, l, force_dense=True)
                    for f in range(2):
                        acc = None
                        for c in range(4):
                            cell = cidx[c]
                            pos = BOFF + cell + (cell & _NM128) + f * 128
                            vals = plsc.load_gather(densev, [pos])
                            term = vals * w4[c]
                            acc = term if acc is None else acc + term
                        encv[2 * l + f, pl.ds(so, 16)] = acc
                for i, l in enumerate(SD):
                    cidx, w4 = corners(xg, yg, l)
                    s = _RES[l] + 1
                    cdel = (1, s, s + 1)
                    accs = [None, None]
                    for tc in range(NT):
                        if three_d:
                            cell0 = cidx[0] + tpiv[2 * l + tc, :] - sblk[i]
                            wt = tpfv[2 * l + tc, :]
                        else:
                            cell0 = cidx[0]
                        for c in range(4):
                            cell = cell0 if c == 0 else cell0 + cdel[c - 1]
                            wc = w4[c] * wt if three_d else w4[c]
                            pos = dloff[i] + cell + (cell & _NM128)
                            for f in range(2):
                                vals = plsc.load_gather(
                                    densev, [pos + f * 128])
                                term = vals * wc
                                accs[f] = (term if accs[f] is None
                                           else accs[f] + term)
                    for f in range(2):
                        encv[2 * l + f, pl.ds(so, 16)] = accs[f]
                for l in SLV:
                    for f in range(2):
                        acc = None
                        for tc in range(NT):
                            r = RMAP[(l, tc)]
                            rb = r * EL + f * 4 * C
                            for c in range(4):
                                w = wv[b, r, pl.ds(c * C + goff, 16)]
                                vals = rowsvs[b][pl.ds(rb + c * C + goff, 16)]
                                term = vals * w
                                acc = term if acc is None else acc + term
                        encv[2 * l + f, pl.ds(so, 16)] = acc
                return carry

            lax.fori_loop(0, G, grp, 0)

        gen_fire(0, 0)

        def pair(i2, carry):
            i = i2 * 2
            gen_fire(i + 1, 1)
            wait_gathers(0)
            pl.when(i2 > 0)(drain_out)
            accum(i, 0, 0)
            pl.when(i2 < NP - 1)(lambda: gen_fire(i + 2, 0))
            wait_gathers(1)
            accum(i + 1, 1, 64)
            pltpu.async_copy(encv, out_hbm.at[bbase + i2], osem)
            return carry

        lax.fori_loop(0, NP, pair, 0)
        drain_out()

    kern = pl.kernel(body,
                     out_type=jax.ShapeDtypeStruct((_N // 128, 32, 128),
                                                   jnp.float32),
                     mesh=mesh, scratch_types=scratch,
                     compiler_params=pltpu.CompilerParams(
                         needs_layout_passes=False))
    if three_d:
        return kern(coords_flat, tpari, tparf, table_flat)
    return kern(coords_flat, table_flat)


def _mlp_call(pe_blocks_in, f2, exrow, enc, w0c, w0e, w1, w2, n_out,
              is_deform):
    BLK = 2048
    grid_steps = _N // BLK
    BR = BLK // 128

    def body(fin_ref, ex_ref, enc_ref, w0c_ref, w0e_ref, w1_ref, w2_ref,
             out_ref):
        fin = fin_ref[...]
        f2b = fin.reshape(BR, 2, 128).swapaxes(1, 2).reshape(BLK, 2)
        encb = enc_ref[...].swapaxes(1, 2).reshape(BLK, 32)
        h = jnp.dot(encb, w0e_ref[...],
                    preferred_element_type=jnp.float32)
        h = h + jnp.dot(f2b, w0c_ref[...],
                        preferred_element_type=jnp.float32)
        h = h + ex_ref[...]
        h = jnp.maximum(h, 0.0)
        h = jnp.maximum(jnp.dot(h, w1_ref[...],
                                preferred_element_type=jnp.float32), 0.0)
        o = jnp.dot(h, w2_ref[...], preferred_element_type=jnp.float32)
        if is_deform:
            pe = (o / 5.0 + f2b + 0.3) / 1.6
            out_ref[...] = pe.reshape(BR, 128, 2).swapaxes(1, 2).reshape(
                2 * BR, 128)
        else:
            out_ref[...] = o

    if is_deform:
        out_shape = jax.ShapeDtypeStruct((_N // 128 * 2, 128), jnp.float32)
        out_spec = pl.BlockSpec((2 * BR, 128), lambda i: (i, 0))
    else:
        out_shape = jax.ShapeDtypeStruct((_N, n_out), jnp.float32)
        out_spec = pl.BlockSpec((BLK, n_out), lambda i: (i, 0))

    fin = pe_blocks_in if pe_blocks_in is not None else f2
    return pl.pallas_call(
        body,
        grid=(grid_steps,),
        in_specs=[
            pl.BlockSpec((2 * BR, 128), lambda i: (i, 0)),
            pl.BlockSpec((1, 64), lambda i: (0, 0)),
            pl.BlockSpec((BR, 32, 128), lambda i: (i, 0, 0)),
            pl.BlockSpec((2, 64), lambda i: (0, 0)),
            pl.BlockSpec((32, 64), lambda i: (0, 0)),
            pl.BlockSpec((64, 64), lambda i: (0, 0)),
            pl.BlockSpec((64, n_out), lambda i: (0, 0)),
        ],
        out_specs=out_spec,
        out_shape=out_shape,
    )(fin, exrow, enc, w0c, w0e, w1, w2)


def kernel(tseq, grid, deform_table, dW0, dW1, dW2, video_table, vW0, vW1,
           vW2):
    gblocks = grid.reshape(_N // 128, 128, 2).swapaxes(1, 2).reshape(
        _N // 128 * 2, 128)
    gflat = gblocks.reshape(-1)
    dtab = deform_table.reshape(_N_LEVELS, _T // 128, 128, 2).swapaxes(
        2, 3).reshape(-1)
    vtab = video_table.reshape(_N_LEVELS, _T // 128, 128, 2).swapaxes(
        2, 3).reshape(-1)

    t = tseq[0, 0]
    cti, ctf = [], []
    for l in range(_N_LEVELS):
        res = _RES[l]
        s = res + 1
        pt = t * res
        it0f = jnp.clip(jnp.floor(pt), 0.0, float(res - 1))
        ft = pt - it0f
        it0 = it0f.astype(jnp.int32)
        it1 = it0 + 1
        if s ** 3 <= _T:
            ct0 = it0 * (s * s)
            ct1 = it1 * (s * s)
        else:
            ct0 = it0 * _P2
            ct1 = it1 * _P2
        cti += [ct0, ct1]
        ctf += [1.0 - ft, ft]
    tpari = jnp.broadcast_to(jnp.stack(cti)[:, None],
                             (2 * _N_LEVELS, 16)).astype(jnp.int32)
    tparf = jnp.broadcast_to(jnp.stack(ctf)[:, None],
                             (2 * _N_LEVELS, 16)).astype(jnp.float32)

    enc1 = _encode_call(gflat, dtab, tpari, tparf, three_d=True)
    exrow = t * dW0[2:3, :]
    pe_blocks = _mlp_call(None, gblocks, exrow, enc1, dW0[:2], dW0[3:],
                          dW1, dW2, 2, is_deform=True)
    enc2 = _encode_call(pe_blocks.reshape(-1), vtab, None, None,
                        three_d=False)
    out = _mlp_call(pe_blocks, None, jnp.zeros((1, 64), jnp.float32), enc2,
                    vW0[:2], vW0[2:], vW1, vW2, 3, is_deform=False)
    return out
